# baseline (speedup 1.0000x reference)
import functools

import jax
import jax.numpy as jnp
from jax import lax
from jax.experimental import pallas as pl
from jax.experimental.pallas import tpu as pltpu
from jax.experimental.pallas import tpu_sc as plsc


def _node_transform_body(x0_ref, x1_ref, w0_ref, w1_ref, b0_ref, b1_ref,
                         y0_ref, y1_ref):
    y0_ref[...] = (
        jnp.dot(x0_ref[...], w0_ref[...], preferred_element_type=jnp.float32)
        + b0_ref[...]
    )
    y1_ref[...] = (
        jnp.dot(x1_ref[...], w1_ref[...], preferred_element_type=jnp.float32)
        + b1_ref[...]
    )


def _node_transform(x0, x1, W0, b0, W1, b1):
    n, d = x0.shape
    grid = 10
    rows = n // grid
    row_spec = pl.BlockSpec((rows, d), lambda i: (i, 0))
    full_spec = pl.BlockSpec((d, d), lambda i: (0, 0))
    bias_spec = pl.BlockSpec((1, d), lambda i: (0, 0))
    return pl.pallas_call(
        _node_transform_body,
        grid=(grid,),
        in_specs=[row_spec, row_spec, full_spec, full_spec, bias_spec,
                  bias_spec],
        out_specs=[row_spec, row_spec],
        out_shape=[
            jax.ShapeDtypeStruct((n, d), jnp.float32),
            jax.ShapeDtypeStruct((n, d), jnp.float32),
        ],
    )(x0, x1, W0, W1, b0.reshape(1, d), b1.reshape(1, d))


_CHUNK = 128
_NSLOT = 6


def _make_gather_add(E, D):
    info = plsc.get_sparse_core_info()
    nw = info.num_cores * info.num_subcores
    e_per_w = E // nw
    assert e_per_w * nw == E and e_per_w % 8 == 0
    n_full = e_per_w // _CHUNK
    tail = e_per_w - n_full * _CHUNK
    assert tail % 8 == 0
    n_steps = n_full + 2
    n_outer = (n_steps + _NSLOT - 1) // _NSLOT
    mesh = plsc.VectorSubcoreMesh(core_axis_name="c", subcore_axis_name="s")

    @functools.partial(
        pl.kernel,
        mesh=mesh,
        out_type=jax.ShapeDtypeStruct((E, D), jnp.float32),
        scratch_types=(
            [pltpu.VMEM((e_per_w,), jnp.int32)] * 2
            + [pltpu.VMEM((_CHUNK, D), jnp.float32)] * _NSLOT
            + [pltpu.SemaphoreType.DMA] * (3 * _NSLOT)
        ),
    )
    def gather_add(y0_hbm, y1_hbm, src0_hbm, src1_hbm, out_hbm,
                   i0_all, i1_all, *bufs_and_sems):
        rbuf = bufs_and_sems[:_NSLOT]
        sem_g = bufs_and_sems[_NSLOT:2 * _NSLOT]
        sem_a = bufs_and_sems[2 * _NSLOT:3 * _NSLOT]
        sem_w = bufs_and_sems[3 * _NSLOT:4 * _NSLOT]
        wid = lax.axis_index("s") * info.num_cores + lax.axis_index("c")
        base = wid * e_per_w

        pltpu.sync_copy(src0_hbm.at[pl.ds(base, e_per_w)], i0_all)
        pltpu.sync_copy(src1_hbm.at[pl.ds(base, e_per_w)], i1_all)

        def idx0(c):
            return i0_all.at[pl.ds(c * _CHUNK, _CHUNK)]

        def idx1(c):
            return i1_all.at[pl.ds(c * _CHUNK, _CHUNK)]

        def out_slice(c):
            return out_hbm.at[pl.ds(base + c * _CHUNK, _CHUNK)]

        def step(outer, k):
            c2 = outer * _NSLOT + k
            cA = c2
            cB = c2 - 1
            cC = c2 - 2
            bA = k
            bB = (k - 1) % _NSLOT
            bC = (k - 2) % _NSLOT

            @pl.when(jnp.logical_and(cC >= 0, cC < n_full))
            def _():
                pltpu.make_async_copy(
                    y1_hbm.at[idx1(cC)], rbuf[bC], sem_a[bC]).wait()
                pltpu.async_copy(rbuf[bC], out_slice(cC), sem_w[bC])

            @pl.when(jnp.logical_and(cB >= 0, cB < n_full))
            def _():
                pltpu.make_async_copy(
                    y0_hbm.at[idx0(cB)], rbuf[bB], sem_g[bB]).wait()
                pltpu.async_copy(y1_hbm.at[idx1(cB)], rbuf[bB], sem_a[bB],
                                 add=True)

            @pl.when(cA < n_full)
            def _():
                @pl.when(cA >= _NSLOT)
                def _():
                    pltpu.make_async_copy(
                        rbuf[bA], out_slice(cA - _NSLOT), sem_w[bA]).wait()
                pltpu.async_copy(y0_hbm.at[idx0(cA)], rbuf[bA], sem_g[bA])

        def outer_body(outer, carry):
            for k in range(_NSLOT):
                step(outer, k)
            return carry

        lax.fori_loop(0, n_outer, outer_body, 0)

        for j in range(_NSLOT):
            c = n_full - _NSLOT + j
            pltpu.make_async_copy(
                rbuf[c % _NSLOT], out_slice(c), sem_w[c % _NSLOT]).wait()

        if tail:
            off = n_full * _CHUNK
            ti0 = i0_all.at[pl.ds(off, tail)]
            ti1 = i1_all.at[pl.ds(off, tail)]
            tb = rbuf[0].at[pl.ds(0, tail)]
            pltpu.async_copy(y0_hbm.at[ti0], tb, sem_g[0]).wait()
            pltpu.sync_copy(y1_hbm.at[ti1], tb, add=True)
            pltpu.sync_copy(tb, out_hbm.at[pl.ds(base + off, tail)])

    return gather_add


def kernel(x0, x1, src0, src1, W0, b0, W1, b1):
    y0, y1 = _node_transform(x0, x1, W0, b0, W1, b1)
    E = src0.shape[0]
    D = x0.shape[1]
    return _make_gather_add(E, D)(y0, y1, src0, src1)


# speedup vs baseline: 1.0394x; 1.0394x over previous
import functools

import jax
import jax.numpy as jnp
from jax import lax
from jax.experimental import pallas as pl
from jax.experimental.pallas import tpu as pltpu
from jax.experimental.pallas import tpu_sc as plsc


def _node_transform_body(x0_ref, x1_ref, w0_ref, w1_ref, b0_ref, b1_ref,
                         y0_ref, y1_ref):
    y0_ref[...] = (
        jnp.dot(x0_ref[...], w0_ref[...], preferred_element_type=jnp.float32)
        + b0_ref[...]
    )
    y1_ref[...] = (
        jnp.dot(x1_ref[...], w1_ref[...], preferred_element_type=jnp.float32)
        + b1_ref[...]
    )


def _node_transform(x0, x1, W0, b0, W1, b1):
    n, d = x0.shape
    grid = 10
    rows = n // grid
    row_spec = pl.BlockSpec((rows, d), lambda i: (i, 0))
    full_spec = pl.BlockSpec((d, d), lambda i: (0, 0))
    bias_spec = pl.BlockSpec((1, d), lambda i: (0, 0))
    return pl.pallas_call(
        _node_transform_body,
        grid=(grid,),
        in_specs=[row_spec, row_spec, full_spec, full_spec, bias_spec,
                  bias_spec],
        out_specs=[row_spec, row_spec],
        out_shape=[
            jax.ShapeDtypeStruct((n, d), jnp.float32),
            jax.ShapeDtypeStruct((n, d), jnp.float32),
        ],
    )(x0, x1, W0, W1, b0.reshape(1, d), b1.reshape(1, d))


_CHUNK = 128
_NSLOT = 6


def _make_gather_add(E, D):
    info = plsc.get_sparse_core_info()
    nw = info.num_cores * info.num_subcores
    e_per_w = E // nw
    assert e_per_w * nw == E and e_per_w % 8 == 0
    n_full = e_per_w // _CHUNK
    tail = e_per_w - n_full * _CHUNK
    assert tail % 8 == 0
    n_steps = n_full + 4
    n_outer = (n_steps + _NSLOT - 1) // _NSLOT
    mesh = plsc.VectorSubcoreMesh(core_axis_name="c", subcore_axis_name="s")

    @functools.partial(
        pl.kernel,
        mesh=mesh,
        out_type=jax.ShapeDtypeStruct((E, D), jnp.float32),
        scratch_types=(
            [pltpu.VMEM((e_per_w,), jnp.int32)] * 2
            + [pltpu.VMEM((_CHUNK, D), jnp.float32)] * _NSLOT
            + [pltpu.SemaphoreType.DMA] * (3 * _NSLOT)
        ),
    )
    def gather_add(y0_hbm, y1_hbm, src0_hbm, src1_hbm, out_hbm,
                   i0_all, i1_all, *bufs_and_sems):
        rbuf = bufs_and_sems[:_NSLOT]
        sem_g = bufs_and_sems[_NSLOT:2 * _NSLOT]
        sem_a = bufs_and_sems[2 * _NSLOT:3 * _NSLOT]
        sem_w = bufs_and_sems[3 * _NSLOT:4 * _NSLOT]
        wid = lax.axis_index("s") * info.num_cores + lax.axis_index("c")
        base = wid * e_per_w

        pltpu.sync_copy(src0_hbm.at[pl.ds(base, e_per_w)], i0_all)
        pltpu.sync_copy(src1_hbm.at[pl.ds(base, e_per_w)], i1_all)

        def idx0(c):
            return i0_all.at[pl.ds(c * _CHUNK, _CHUNK)]

        def idx1(c):
            return i1_all.at[pl.ds(c * _CHUNK, _CHUNK)]

        def out_slice(c):
            return out_hbm.at[pl.ds(base + c * _CHUNK, _CHUNK)]

        def step(outer, k):
            c2 = outer * _NSLOT + k
            cA = c2
            cB = c2 - 2
            cC = c2 - 4
            bA = k
            bB = (k - 2) % _NSLOT
            bC = (k - 4) % _NSLOT

            @pl.when(jnp.logical_and(cC >= 0, cC < n_full))
            def _():
                pltpu.make_async_copy(
                    y1_hbm.at[idx1(cC)], rbuf[bC], sem_a[bC]).wait()
                pltpu.async_copy(rbuf[bC], out_slice(cC), sem_w[bC])

            @pl.when(jnp.logical_and(cB >= 0, cB < n_full))
            def _():
                pltpu.make_async_copy(
                    y0_hbm.at[idx0(cB)], rbuf[bB], sem_g[bB]).wait()
                pltpu.async_copy(y1_hbm.at[idx1(cB)], rbuf[bB], sem_a[bB],
                                 add=True)

            @pl.when(cA < n_full)
            def _():
                @pl.when(cA >= _NSLOT)
                def _():
                    pltpu.make_async_copy(
                        rbuf[bA], out_slice(cA - _NSLOT), sem_w[bA]).wait()
                pltpu.async_copy(y0_hbm.at[idx0(cA)], rbuf[bA], sem_g[bA])

        def outer_body(outer, carry):
            for k in range(_NSLOT):
                step(outer, k)
            return carry

        lax.fori_loop(0, n_outer, outer_body, 0)

        for j in range(_NSLOT):
            c = n_full - _NSLOT + j
            pltpu.make_async_copy(
                rbuf[c % _NSLOT], out_slice(c), sem_w[c % _NSLOT]).wait()

        if tail:
            off = n_full * _CHUNK
            ti0 = i0_all.at[pl.ds(off, tail)]
            ti1 = i1_all.at[pl.ds(off, tail)]
            tb = rbuf[0].at[pl.ds(0, tail)]
            pltpu.async_copy(y0_hbm.at[ti0], tb, sem_g[0]).wait()
            pltpu.sync_copy(y1_hbm.at[ti1], tb, add=True)
            pltpu.sync_copy(tb, out_hbm.at[pl.ds(base + off, tail)])

    return gather_add


def kernel(x0, x1, src0, src1, W0, b0, W1, b1):
    y0, y1 = _node_transform(x0, x1, W0, b0, W1, b1)
    E = src0.shape[0]
    D = x0.shape[1]
    return _make_gather_add(E, D)(y0, y1, src0, src1)
